# baseline (speedup 1.0000x reference)
import functools

import jax
import jax.numpy as jnp
from jax import lax
from jax.experimental import pallas as pl
from jax.experimental.pallas import tpu as pltpu
from jax.experimental.pallas import tpu_sc as plsc

_NC = 2
_NS = 16
_NW = _NC * _NS
_L = 16
_SUB = 64
_GW = 4
_PW = 512


def _pack_block(in_ref, out_ref):
    x = in_ref[...]
    y = x.astype(jnp.bfloat16)
    u = pltpu.bitcast(y, jnp.int32)
    t = jnp.swapaxes(u, 0, 1)
    q = _PW // _GW
    out_ref[...] = jnp.concatenate(
        [t[s * q:(s + 1) * q, :] for s in range(_GW)], axis=1)


@functools.lru_cache(maxsize=None)
def _build_pack(V, D):
    @jax.jit
    def pack(tab):
        tab_t = jnp.transpose(tab)
        return pl.pallas_call(
            _pack_block,
            grid=(V // _PW,),
            in_specs=[pl.BlockSpec((D, _PW), lambda i: (0, i))],
            out_specs=pl.BlockSpec((_PW // _GW, _GW * D // 2),
                                   lambda i: (i, 0)),
            out_shape=jax.ShapeDtypeStruct((V // _GW, _GW * D // 2),
                                           jnp.int32),
        )(tab_t)
    return pack


@functools.lru_cache(maxsize=None)
def _build(B, C, D, V):
    BPW = B // _NW
    SUB = _SUB
    NSUB = BPW // SUB
    CR = SUB * C
    KW = D // 2
    NBUF = 2
    NEB = CR // _L
    DPAD = -(-CR // 128) * 128

    mesh = plsc.VectorSubcoreMesh(
        core_axis_name="c", subcore_axis_name="s",
        num_cores=_NC, num_subcores=_NS)

    @functools.partial(
        pl.kernel,
        out_type=jax.ShapeDtypeStruct((B // SUB * DPAD,), jnp.float32),
        mesh=mesh,
        compiler_params=pltpu.CompilerParams(
            needs_layout_passes=False, use_tc_tiling_on_sc=True),
        scratch_types=[
            pltpu.VMEM((NSUB, SUB), jnp.int32),
            pltpu.VMEM((NSUB, SUB), jnp.int32),
            pltpu.VMEM((NSUB * C, SUB), jnp.int32),
            pltpu.VMEM((NSUB * C, SUB), jnp.int32),
            pltpu.VMEM((NBUF, SUB, _GW * 32), jnp.int32),
            pltpu.VMEM((NBUF, CR, _GW * 32), jnp.int32),
            pltpu.VMEM((NBUF, DPAD), jnp.float32),
            pltpu.SemaphoreType.DMA,
            pltpu.SemaphoreType.DMA,
            pltpu.SemaphoreType.DMA,
            pltpu.SemaphoreType.DMA,
        ],
    )
    def sckern(tg_hbm, ts_hbm, cg_hbm, cs_hbm, tt_hbm, ct_hbm, out_hbm,
               tgid, tsub, cgid, csub, tgrp, cgrp, dots,
               sem_i, sem_g0, sem_g1, sem_o):
        cid = lax.axis_index("c")
        sid = lax.axis_index("s")
        wid = sid * _NC + cid
        gsems = [sem_g0, sem_g1]

        idescs = [
            pltpu.async_copy(
                tg_hbm.at[pl.ds(wid * NSUB, NSUB)], tgid, sem_i),
            pltpu.async_copy(
                ts_hbm.at[pl.ds(wid * NSUB, NSUB)], tsub, sem_i),
            pltpu.async_copy(
                cg_hbm.at[pl.ds(wid * NSUB * C, NSUB * C)], cgid, sem_i),
            pltpu.async_copy(
                cs_hbm.at[pl.ds(wid * NSUB * C, NSUB * C)], csub, sem_i),
        ]
        for d in idescs:
            d.wait()

        def gather_descs(j, buf):
            descs = [pltpu.make_async_copy(
                tt_hbm.at[tgid.at[j]], tgrp.at[buf], gsems[buf])]
            for k in range(C):
                descs.append(pltpu.make_async_copy(
                    ct_hbm.at[cgid.at[j * C + k]],
                    cgrp.at[buf, pl.ds(k * SUB, SUB)], gsems[buf]))
            return descs

        def out_desc(j, buf):
            return pltpu.make_async_copy(
                dots.at[buf],
                out_hbm.at[pl.ds((wid * NSUB + j) * DPAD, DPAD)], sem_o)

        iota = lax.iota(jnp.int32, _L)
        magic = (1 << 18) // C + 1

        def compute(j, buf):
            @pl.loop(0, NEB)
            def _eb(eb):
                e_vec = eb * _L + iota
                b_vec = (e_vec * magic) >> 18
                c_vec = e_vec - b_vec * C
                crow = c_vec * SUB + b_vec
                cd = plsc.load_gather(csub, [j * C + c_vec, b_vec]) * KW
                td = plsc.load_gather(tsub, [iota * 0 + j, b_vec]) * KW
                acc_e = jnp.zeros((_L,), jnp.float32)
                acc_o = jnp.zeros((_L,), jnp.float32)
                for k in range(KW):
                    xk = plsc.load_gather(
                        cgrp, [iota * 0 + buf, crow, cd + k])
                    wk = plsc.load_gather(
                        tgrp, [iota * 0 + buf, b_vec, td + k])
                    xe, xo = plsc.unpack(plsc.bitcast(xk, jnp.bfloat16),
                                         format=plsc.PackFormat.INTERLEAVED)
                    we, wo = plsc.unpack(plsc.bitcast(wk, jnp.bfloat16),
                                         format=plsc.PackFormat.INTERLEAVED)
                    acc_e = acc_e + xe * we
                    acc_o = acc_o + xo * wo
                dots[buf, pl.ds(eb * _L, _L)] = acc_e + acc_o

        for rb in range(NBUF):
            for d in gather_descs(rb, rb):
                d.start()

        @pl.loop(0, NSUB, step=NBUF)
        def _sub(j0):
            for rb in range(NBUF):
                j = j0 + rb
                for d in gather_descs(j, rb):
                    d.wait()

                @pl.when(j >= NBUF)
                def _():
                    out_desc(j - NBUF, rb).wait()

                compute(j, rb)
                out_desc(j, rb).start()

                @pl.when(j + NBUF < NSUB)
                def _():
                    for d in gather_descs(j + NBUF, rb):
                        d.start()

        for rb in range(NBUF):
            out_desc(NSUB - NBUF + rb, rb).wait()

    return sckern


def kernel(target, context, target_table, context_table):
    B, C = context.shape
    V, D = target_table.shape
    sck = _build(B, C, D, V)
    pack = _build_pack(V, D)

    tgt = target.astype(jnp.int32)
    ctx4 = (context.astype(jnp.int32)
            .reshape(_NW, -1, _SUB, C)
            .transpose(0, 1, 3, 2)
            .reshape(-1, _SUB))
    tg2 = tgt.reshape(-1, _SUB)

    q = _PW // _GW

    def gid(w):
        return (w // _PW) * q + (w % q)

    def sid(w):
        return (w % _PW) // q

    out = sck(gid(tg2), sid(tg2), gid(ctx4), sid(ctx4),
              pack(target_table), pack(context_table))
    sub_c = _SUB * C
    dpad = -(-sub_c // 128) * 128
    return out.reshape(-1, dpad)[:, :sub_c].reshape(B, C)
